# baseline (speedup 1.0000x reference)
import functools

import jax
import jax.numpy as jnp
from jax import lax
from jax.experimental import pallas as pl
from jax.experimental.pallas import tpu as pltpu
from jax.experimental.pallas import tpu_sc as plsc

_NUM_CORES = 1
_NUM_SUBCORES = 16
_NW = _NUM_CORES * _NUM_SUBCORES
_LANES = 16
_NBINS = 128
_ROWSTRIDE = 129
_HISTWORDS = _LANES * _ROWSTRIDE + _LANES


def _sc_hist_body(main_vecs, rem_vecs, halt_hbm, resp_hbm, out_hbm,
                  halt_v, resp_v, tail_h, tail_r,
                  hist_halt, hist_resp, out_v, sem_h, sem_r):
    c = lax.axis_index("c")
    s = lax.axis_index("s")
    wid = s * _NUM_CORES + c
    chunk = main_vecs * _LANES
    base = wid * chunk

    cp_h = pltpu.async_copy(halt_hbm.at[pl.ds(base, chunk)], halt_v, sem_h)
    cp_r = pltpu.async_copy(resp_hbm.at[pl.ds(base, chunk)], resp_v, sem_r)

    lane_base = lax.iota(jnp.int32, _LANES) * _ROWSTRIDE
    ones = jnp.full((_LANES,), 1.0, jnp.float32)
    zeros = jnp.zeros((_LANES,), jnp.float32)

    def zbody(i, carry):
        hist_halt[pl.ds(i * _LANES, _LANES)] = zeros
        hist_resp[pl.ds(i * _LANES, _LANES)] = zeros
        return carry

    lax.fori_loop(0, _HISTWORDS // _LANES, zbody, 0, unroll=2)

    cp_h.wait()
    cp_r.wait()

    def body(i, carry):
        v = halt_v[pl.ds(i * _LANES, _LANES)]
        plsc.addupdate_scatter(hist_halt, [lane_base + v], ones)
        w = resp_v[pl.ds(i * _LANES, _LANES)]
        plsc.addupdate_scatter(hist_resp, [lane_base + w], ones)
        return carry

    lax.fori_loop(0, main_vecs, body, 0, unroll=4)

    if rem_vecs:
        @pl.when(wid < rem_vecs)
        def _():
            tail_base = _NW * chunk + wid * _LANES
            pltpu.sync_copy(halt_hbm.at[pl.ds(tail_base, _LANES)], tail_h)
            pltpu.sync_copy(resp_hbm.at[pl.ds(tail_base, _LANES)], tail_r)
            plsc.addupdate_scatter(hist_halt, [lane_base + tail_h[...]], ones)
            plsc.addupdate_scatter(hist_resp, [lane_base + tail_r[...]], ones)

    def fold_body(cc, carry):
        def racc(r, accs):
            ah, ar = accs
            off = r * _ROWSTRIDE + cc * _LANES
            ah = ah + hist_halt[pl.ds(off, _LANES)]
            ar = ar + hist_resp[pl.ds(off, _LANES)]
            return (ah, ar)

        ah, ar = lax.fori_loop(0, _LANES, racc, (zeros, zeros), unroll=8)
        out_v[0, pl.ds(cc * _LANES, _LANES)] = ah
        out_v[1, pl.ds(cc * _LANES, _LANES)] = ar
        return carry

    lax.fori_loop(0, _NBINS // _LANES, fold_body, 0)

    pltpu.sync_copy(out_v, out_hbm.at[wid])


def _kl_body(steps, parts_ref, out_ref):
    h = parts_ref[...]
    tot = jnp.sum(h, axis=0)
    pred = tot[0:1, :]
    true_ = tot[1:2, :]
    col = lax.broadcasted_iota(jnp.int32, (1, _NBINS), 1)
    m = jnp.logical_and(col >= 1, col <= steps)
    pred = jnp.where(m, pred, 0.0)
    true_ = jnp.where(m, true_, 0.0)
    pred = pred / jnp.maximum(jnp.sum(pred), 1e-12)
    true_ = true_ / jnp.maximum(jnp.sum(true_), 1e-12)
    safe = jnp.where(true_ > 0, true_, 1.0)
    tlogt = jnp.where(true_ > 0, true_ * jnp.log(safe), 0.0)
    kl = jnp.sum(tlogt - true_ * pred, keepdims=True).reshape(1, 1)
    out_ref[...] = kl / jnp.float32(steps)


def kernel(p_halts, halt_steps, trial_types, response_steps):
    steps = p_halts.shape[1]
    n = halt_steps.shape[0]

    nvec = n // _LANES
    leftover = n - nvec * _LANES
    if leftover:
        pad = _LANES - leftover
        zpad = jnp.zeros((pad,), halt_steps.dtype)
        halt_steps = jnp.concatenate([halt_steps, zpad])
        response_steps = jnp.concatenate([response_steps, zpad])
        nvec += 1
    main_vecs = nvec // _NW
    rem_vecs = nvec - main_vecs * _NW
    chunk = main_vecs * _LANES

    mesh = plsc.VectorSubcoreMesh(
        core_axis_name="c", subcore_axis_name="s",
        num_cores=_NUM_CORES, num_subcores=_NUM_SUBCORES)

    hist_call = pl.kernel(
        functools.partial(_sc_hist_body, main_vecs, rem_vecs),
        out_type=jax.ShapeDtypeStruct((_NW, 2, _NBINS), jnp.float32),
        mesh=mesh,
        scratch_types=[
            pltpu.VMEM((chunk,), jnp.int32),
            pltpu.VMEM((chunk,), jnp.int32),
            pltpu.VMEM((_LANES,), jnp.int32),
            pltpu.VMEM((_LANES,), jnp.int32),
            pltpu.VMEM((_HISTWORDS,), jnp.float32),
            pltpu.VMEM((_HISTWORDS,), jnp.float32),
            pltpu.VMEM((2, _NBINS), jnp.float32),
            pltpu.SemaphoreType.DMA,
            pltpu.SemaphoreType.DMA,
        ],
        compiler_params=pltpu.CompilerParams(needs_layout_passes=False),
    )
    parts = hist_call(halt_steps, response_steps)

    loss2d = pl.pallas_call(
        functools.partial(_kl_body, steps),
        out_shape=jax.ShapeDtypeStruct((1, 1), jnp.float32),
    )(parts)
    return loss2d[0, 0]


# speedup vs baseline: 1.0653x; 1.0653x over previous
import functools

import jax
import jax.numpy as jnp
from jax import lax
from jax.experimental import pallas as pl
from jax.experimental.pallas import tpu as pltpu
from jax.experimental.pallas import tpu_sc as plsc

_NUM_CORES = 2
_NUM_SUBCORES = 16
_NW = _NUM_CORES * _NUM_SUBCORES
_LANES = 16
_NBINS = 128
_ROWSTRIDE = 129
_HISTWORDS = _LANES * _ROWSTRIDE + _LANES


def _sc_hist_body(main_vecs, rem_vecs, halt_hbm, resp_hbm,
                  out_halt_hbm, out_resp_hbm,
                  halt_v, resp_v, tail_h, tail_r,
                  hist_halt, hist_resp, out_h, out_r, sem_h, sem_r):
    c = lax.axis_index("c")
    s = lax.axis_index("s")
    wid = s * _NUM_CORES + c
    chunk = main_vecs * _LANES
    base = wid * chunk

    cp_h = pltpu.async_copy(halt_hbm.at[pl.ds(base, chunk)], halt_v, sem_h)
    cp_r = pltpu.async_copy(resp_hbm.at[pl.ds(base, chunk)], resp_v, sem_r)

    lane_base = lax.iota(jnp.int32, _LANES) * _ROWSTRIDE
    ones = jnp.full((_LANES,), 1.0, jnp.float32)
    zeros = jnp.zeros((_LANES,), jnp.float32)

    def zbody(i, carry):
        hist_halt[pl.ds(i * _LANES, _LANES)] = zeros
        hist_resp[pl.ds(i * _LANES, _LANES)] = zeros
        return carry

    lax.fori_loop(0, _HISTWORDS // _LANES, zbody, 0, unroll=2)

    cp_h.wait()
    cp_r.wait()

    def body(i, carry):
        v = halt_v[pl.ds(i * _LANES, _LANES)]
        plsc.addupdate_scatter(hist_halt, [lane_base + v], ones)
        w = resp_v[pl.ds(i * _LANES, _LANES)]
        plsc.addupdate_scatter(hist_resp, [lane_base + w], ones)
        return carry

    lax.fori_loop(0, main_vecs, body, 0, unroll=4)

    if rem_vecs:
        @pl.when(wid < rem_vecs)
        def _():
            tail_base = _NW * chunk + wid * _LANES
            pltpu.sync_copy(halt_hbm.at[pl.ds(tail_base, _LANES)], tail_h)
            pltpu.sync_copy(resp_hbm.at[pl.ds(tail_base, _LANES)], tail_r)
            plsc.addupdate_scatter(hist_halt, [lane_base + tail_h[...]], ones)
            plsc.addupdate_scatter(hist_resp, [lane_base + tail_r[...]], ones)

    def fold_body(cc, carry):
        def racc(r, accs):
            ah, ar = accs
            off = r * _ROWSTRIDE + cc * _LANES
            ah = ah + hist_halt[pl.ds(off, _LANES)]
            ar = ar + hist_resp[pl.ds(off, _LANES)]
            return (ah, ar)

        ah, ar = lax.fori_loop(0, _LANES, racc, (zeros, zeros), unroll=8)
        out_h[pl.ds(cc * _LANES, _LANES)] = ah
        out_r[pl.ds(cc * _LANES, _LANES)] = ar
        return carry

    lax.fori_loop(0, _NBINS // _LANES, fold_body, 0)

    pltpu.sync_copy(out_h, out_halt_hbm.at[wid])
    pltpu.sync_copy(out_r, out_resp_hbm.at[wid])


def _kl_body(steps, halt_parts_ref, resp_parts_ref, out_ref):
    pred = jnp.sum(halt_parts_ref[...], axis=0, keepdims=True)
    true_ = jnp.sum(resp_parts_ref[...], axis=0, keepdims=True)
    col = lax.broadcasted_iota(jnp.int32, (1, _NBINS), 1)
    m = jnp.logical_and(col >= 1, col <= steps)
    pred = jnp.where(m, pred, 0.0)
    true_ = jnp.where(m, true_, 0.0)
    pred = pred / jnp.maximum(jnp.sum(pred), 1e-12)
    true_ = true_ / jnp.maximum(jnp.sum(true_), 1e-12)
    safe = jnp.where(true_ > 0, true_, 1.0)
    tlogt = jnp.where(true_ > 0, true_ * jnp.log(safe), 0.0)
    kl = jnp.sum(tlogt - true_ * pred, keepdims=True).reshape(1, 1)
    out_ref[...] = kl / jnp.float32(steps)


def kernel(p_halts, halt_steps, trial_types, response_steps):
    steps = p_halts.shape[1]
    n = halt_steps.shape[0]

    nvec = n // _LANES
    leftover = n - nvec * _LANES
    if leftover:
        pad = _LANES - leftover
        zpad = jnp.zeros((pad,), halt_steps.dtype)
        halt_steps = jnp.concatenate([halt_steps, zpad])
        response_steps = jnp.concatenate([response_steps, zpad])
        nvec += 1
    main_vecs = nvec // _NW
    rem_vecs = nvec - main_vecs * _NW
    chunk = main_vecs * _LANES

    mesh = plsc.VectorSubcoreMesh(
        core_axis_name="c", subcore_axis_name="s",
        num_cores=_NUM_CORES, num_subcores=_NUM_SUBCORES)

    hist_call = pl.kernel(
        functools.partial(_sc_hist_body, main_vecs, rem_vecs),
        out_type=(jax.ShapeDtypeStruct((_NW, _NBINS), jnp.float32),
                  jax.ShapeDtypeStruct((_NW, _NBINS), jnp.float32)),
        mesh=mesh,
        scratch_types=[
            pltpu.VMEM((chunk,), jnp.int32),
            pltpu.VMEM((chunk,), jnp.int32),
            pltpu.VMEM((_LANES,), jnp.int32),
            pltpu.VMEM((_LANES,), jnp.int32),
            pltpu.VMEM((_HISTWORDS,), jnp.float32),
            pltpu.VMEM((_HISTWORDS,), jnp.float32),
            pltpu.VMEM((_NBINS,), jnp.float32),
            pltpu.VMEM((_NBINS,), jnp.float32),
            pltpu.SemaphoreType.DMA,
            pltpu.SemaphoreType.DMA,
        ],
        compiler_params=pltpu.CompilerParams(needs_layout_passes=False),
    )
    halt_parts, resp_parts = hist_call(halt_steps, response_steps)

    loss2d = pl.pallas_call(
        functools.partial(_kl_body, steps),
        out_shape=jax.ShapeDtypeStruct((1, 1), jnp.float32),
    )(halt_parts, resp_parts)
    return loss2d[0, 0]
